# baseline (speedup 1.0000x reference)
import functools

import jax
import jax.numpy as jnp
from jax import lax
from jax.experimental import pallas as pl
from jax.experimental.pallas import tpu as pltpu
from jax.experimental.pallas import tpu_sc as plsc

_D = 64
_SCALE = 8.0
_NBATCH = 16384
_SEQ = 50
_SEQP = 56
_DP = 128
_NW = 32
_BATCH_PER_W = _NBATCH // _NW
_CB = 8
_CHUNKS_PER_W = _BATCH_PER_W // _CB
_PAIRS = _CHUNKS_PER_W // 2


@functools.partial(
    pl.kernel,
    mesh=plsc.VectorSubcoreMesh(core_axis_name="c", subcore_axis_name="s"),
    out_type=jax.ShapeDtypeStruct((_NBATCH, _SEQP, _DP), jnp.float32),
    scratch_types=[
        pltpu.VMEM((_BATCH_PER_W, _SEQ), jnp.int32),
        pltpu.VMEM((_CB, _SEQ, _D), jnp.float32),
        pltpu.VMEM((_CB, _SEQ, _D), jnp.float32),
        pltpu.SemaphoreType.DMA,
        pltpu.SemaphoreType.DMA,
        pltpu.SemaphoreType.DMA,
        pltpu.SemaphoreType.DMA,
    ],
    compiler_params=pltpu.CompilerParams(
        needs_layout_passes=False, use_tc_tiling_on_sc=False
    ),
)
def _emb_lookup(idx_hbm, table_hbm, out_hbm, idx_v, rows0, rows1,
                gsem0, gsem1, wsem0, wsem1):
    nc = 2
    wid = lax.axis_index("s") * nc + lax.axis_index("c")
    batch_base = wid * _BATCH_PER_W

    pltpu.sync_copy(idx_hbm.at[pl.ds(batch_base, _BATCH_PER_W)], idx_v)

    def start_gather(ch, buf, sem):
        for j in range(_CB):
            pltpu.async_copy(
                table_hbm.at[idx_v.at[ch * _CB + j]], buf.at[j], sem
            )

    def wait_gather(buf, sem):
        for j in range(_CB):
            pltpu.make_async_copy(
                table_hbm.at[idx_v.at[0]], buf.at[j], sem
            ).wait()

    def start_writeout(ch, buf, sem):
        pltpu.async_copy(
            buf,
            out_hbm.at[
                pl.ds(batch_base + ch * _CB, _CB), pl.ds(0, _SEQ), pl.ds(0, _D)
            ],
            sem,
        )

    def wait_writeout(buf, sem):
        pltpu.make_async_copy(
            buf,
            out_hbm.at[
                pl.ds(batch_base, _CB), pl.ds(0, _SEQ), pl.ds(0, _D)
            ],
            sem,
        ).wait()

    start_gather(0, rows0, gsem0)

    def pair_body(i, _):
        a = 2 * i
        b = a + 1

        @pl.when(i > 0)
        def _():
            wait_writeout(rows1, wsem1)

        start_gather(b, rows1, gsem1)

        wait_gather(rows0, gsem0)
        start_writeout(a, rows0, wsem0)

        wait_gather(rows1, gsem1)
        wait_writeout(rows0, wsem0)

        @pl.when(i < _PAIRS - 1)
        def _():
            start_gather(a + 2, rows0, gsem0)

        start_writeout(b, rows1, wsem1)
        return 0

    lax.fori_loop(0, _PAIRS, pair_body, 0)
    wait_writeout(rows1, wsem1)


def kernel(x, table):
    idx = x.astype(jnp.int32)
    out = _emb_lookup(idx, table)
    f = jnp.where(idx == 0, jnp.float32(0.0), jnp.float32(_SCALE))
    return lax.slice(out, (0, 0, 0), (_NBATCH, _SEQ, _D)) * f[:, :, None]


# speedup vs baseline: 1.1448x; 1.1448x over previous
import functools

import jax
import jax.numpy as jnp
from jax import lax
from jax.experimental import pallas as pl
from jax.experimental.pallas import tpu as pltpu
from jax.experimental.pallas import tpu_sc as plsc

_D = 64
_SCALE = 8.0
_NBATCH = 16384
_SEQ = 50
_SEQP = 56
_DP = 128
_NW = 32
_BATCH_PER_W = _NBATCH // _NW
_CB = 16
_CHUNKS_PER_W = _BATCH_PER_W // _CB
_PAIRS = _CHUNKS_PER_W // 2


@functools.partial(
    pl.kernel,
    mesh=plsc.VectorSubcoreMesh(core_axis_name="c", subcore_axis_name="s"),
    out_type=jax.ShapeDtypeStruct((_NBATCH, _SEQP, _DP), jnp.float32),
    scratch_types=[
        pltpu.VMEM((_BATCH_PER_W, _SEQ), jnp.int32),
        pltpu.VMEM((_CB, _SEQ, _D), jnp.float32),
        pltpu.VMEM((_CB, _SEQ, _D), jnp.float32),
        pltpu.SemaphoreType.DMA,
        pltpu.SemaphoreType.DMA,
        pltpu.SemaphoreType.DMA,
        pltpu.SemaphoreType.DMA,
    ],
    compiler_params=pltpu.CompilerParams(
        needs_layout_passes=False, use_tc_tiling_on_sc=False
    ),
)
def _emb_lookup(idx_hbm, table_hbm, out_hbm, idx_v, rows0, rows1,
                gsem0, gsem1, wsem0, wsem1):
    nc = 2
    wid = lax.axis_index("s") * nc + lax.axis_index("c")
    batch_base = wid * _BATCH_PER_W

    pltpu.sync_copy(idx_hbm.at[pl.ds(batch_base, _BATCH_PER_W)], idx_v)

    def start_gather(ch, buf, sem):
        for j in range(_CB):
            pltpu.async_copy(
                table_hbm.at[idx_v.at[ch * _CB + j]], buf.at[j], sem
            )

    def wait_gather(buf, sem):
        for j in range(_CB):
            pltpu.make_async_copy(
                table_hbm.at[idx_v.at[0]], buf.at[j], sem
            ).wait()

    def start_writeout(ch, buf, sem):
        pltpu.async_copy(
            buf,
            out_hbm.at[
                pl.ds(batch_base + ch * _CB, _CB), pl.ds(0, _SEQ), pl.ds(0, _D)
            ],
            sem,
        )

    def wait_writeout(buf, sem):
        pltpu.make_async_copy(
            buf,
            out_hbm.at[
                pl.ds(batch_base, _CB), pl.ds(0, _SEQ), pl.ds(0, _D)
            ],
            sem,
        ).wait()

    def compute(ch, buf):
        for j in range(_CB):
            jg = ch * _CB + j

            def row_body(r, _, j=j, jg=jg):
                iv = plsc.load_gather(
                    idx_v,
                    [jnp.full((16,), jg, jnp.int32),
                     jnp.full((16,), r, jnp.int32)],
                )
                f = jnp.where(iv == 0, 0.0, _SCALE)
                for c in range(_D // 16):
                    buf[j, r, pl.ds(c * 16, 16)] = (
                        buf[j, r, pl.ds(c * 16, 16)] * f
                    )
                return 0

            lax.fori_loop(0, _SEQ, row_body, 0)

    start_gather(0, rows0, gsem0)

    def pair_body(i, _):
        a = 2 * i
        b = a + 1

        @pl.when(i > 0)
        def _():
            wait_writeout(rows1, wsem1)

        start_gather(b, rows1, gsem1)

        wait_gather(rows0, gsem0)
        compute(a, rows0)
        start_writeout(a, rows0, wsem0)

        wait_gather(rows1, gsem1)
        wait_writeout(rows0, wsem0)

        @pl.when(i < _PAIRS - 1)
        def _():
            start_gather(a + 2, rows0, gsem0)

        compute(b, rows1)
        start_writeout(b, rows1, wsem1)
        return 0

    lax.fori_loop(0, _PAIRS, pair_body, 0)
    wait_writeout(rows1, wsem1)


def kernel(x, table):
    idx = x.astype(jnp.int32)
    out = _emb_lookup(idx, table)
    return lax.slice(out, (0, 0, 0), (_NBATCH, _SEQ, _D))


# speedup vs baseline: 1.2491x; 1.0911x over previous
import functools

import jax
import jax.numpy as jnp
from jax import lax
from jax.experimental import pallas as pl
from jax.experimental.pallas import tpu as pltpu
from jax.experimental.pallas import tpu_sc as plsc

_D = 64
_SCALE = 8.0
_NBATCH = 16384
_SEQ = 50
_SEQP = 56
_DP = 128
_NW = 32
_BATCH_PER_W = _NBATCH // _NW
_CB = 8
_CHUNKS_PER_W = _BATCH_PER_W // _CB
_PAIRS = _CHUNKS_PER_W // 2


@functools.partial(
    pl.kernel,
    mesh=plsc.VectorSubcoreMesh(core_axis_name="c", subcore_axis_name="s"),
    out_type=jax.ShapeDtypeStruct((_NBATCH, _SEQP, _DP), jnp.float32),
    scratch_types=[
        pltpu.VMEM((_BATCH_PER_W, _SEQ), jnp.int32),
        pltpu.VMEM((_CB, _SEQ, _D), jnp.float32),
        pltpu.VMEM((_CB, _SEQ, _D), jnp.float32),
        pltpu.SemaphoreType.DMA,
        pltpu.SemaphoreType.DMA,
        pltpu.SemaphoreType.DMA,
        pltpu.SemaphoreType.DMA,
    ],
    compiler_params=pltpu.CompilerParams(
        needs_layout_passes=False, use_tc_tiling_on_sc=False
    ),
)
def _emb_lookup(idx_hbm, table_hbm, out_hbm, idx_v, rows0, rows1,
                gsem0, gsem1, wsem0, wsem1):
    nc = 2
    wid = lax.axis_index("s") * nc + lax.axis_index("c")
    batch_base = wid * _BATCH_PER_W

    pltpu.sync_copy(idx_hbm.at[pl.ds(batch_base, _BATCH_PER_W)], idx_v)

    def start_gather(ch, buf, sem):
        for j in range(_CB):
            pltpu.async_copy(
                table_hbm.at[idx_v.at[ch * _CB + j]], buf.at[j], sem
            )

    def wait_gather(buf, sem):
        for j in range(_CB):
            pltpu.make_async_copy(
                table_hbm.at[idx_v.at[0]], buf.at[j], sem
            ).wait()

    def start_writeout(ch, buf, sem):
        pltpu.async_copy(
            buf,
            out_hbm.at[
                pl.ds(batch_base + ch * _CB, _CB), pl.ds(0, _SEQ), pl.ds(0, _D)
            ],
            sem,
        )

    def wait_writeout(buf, sem):
        pltpu.make_async_copy(
            buf,
            out_hbm.at[
                pl.ds(batch_base, _CB), pl.ds(0, _SEQ), pl.ds(0, _D)
            ],
            sem,
        ).wait()

    def compute(ch, buf):
        for j in range(_CB):
            jg = ch * _CB + j

            @plsc.parallel_loop(0, _SEQ, step=2, unroll=5)
            def row_body(r, j=j, jg=jg):
                for u in range(2):
                    iv = plsc.load_gather(
                        idx_v,
                        [jnp.full((16,), jg, jnp.int32),
                         jnp.full((16,), u, jnp.int32) + r],
                    )
                    f = jnp.where(iv == 0, 0.0, _SCALE)
                    for c in range(_D // 16):
                        buf[j, r + u, pl.ds(c * 16, 16)] = (
                            buf[j, r + u, pl.ds(c * 16, 16)] * f
                        )

    start_gather(0, rows0, gsem0)

    def pair_body(i, _):
        a = 2 * i
        b = a + 1

        @pl.when(i > 0)
        def _():
            wait_writeout(rows1, wsem1)

        start_gather(b, rows1, gsem1)

        wait_gather(rows0, gsem0)
        compute(a, rows0)
        start_writeout(a, rows0, wsem0)

        wait_gather(rows1, gsem1)
        wait_writeout(rows0, wsem0)

        @pl.when(i < _PAIRS - 1)
        def _():
            start_gather(a + 2, rows0, gsem0)

        compute(b, rows1)
        start_writeout(b, rows1, wsem1)
        return 0

    lax.fori_loop(0, _PAIRS, pair_body, 0)
    wait_writeout(rows1, wsem1)


def kernel(x, table):
    idx = x.astype(jnp.int32)
    out = _emb_lookup(idx, table)
    return lax.slice(out, (0, 0, 0), (_NBATCH, _SEQ, _D))
